# baseline (speedup 1.0000x reference)
import functools

import jax
import jax.numpy as jnp
from jax.experimental import pallas as pl
from jax.experimental.pallas import tpu as pltpu

_TOP_K = 5


def _pass1_kernel(x_ref, s_ref):
    x = x_ref[0]
    e = jnp.exp(x)
    tot = jnp.sum(e, axis=0)
    p = e * (1.0 / tot)
    part = jnp.sum(p, axis=(1, 2))
    part = part.reshape(1, 1, -1)

    @pl.when(pl.program_id(1) == 0)
    def _init():
        s_ref[...] = part

    @pl.when(pl.program_id(1) != 0)
    def _acc():
        s_ref[...] += part


def _top5(x):
    vals = []
    iota = jax.lax.broadcasted_iota(jnp.int32, x.shape, 1)
    for _ in range(_TOP_K):
        m = jnp.max(x, axis=1, keepdims=True)
        vals.append(m)
        idx = jnp.min(jnp.where(x == m, iota, 2**30), axis=1, keepdims=True)
        x = jnp.where(iota == idx, -jnp.inf, x)
    return jnp.concatenate(vals, axis=1), vals[-1]


def _mask_kernel(inv_hw, s_ref, d_ref, r_ref, ksm_ref, klab_ref, kalt_ref):
    sd = s_ref[:, 0, :] * inv_hw
    dist = d_ref[...]
    _, thresh = _top5(sd)
    mask = (sd > thresh).astype(jnp.float32)
    r_ref[...] = ((dist - sd) * mask)[:, None, :]

    a = sd * mask
    top_a, _ = _top5(a)
    ksm_ref[...] = top_a / jnp.sum(a, axis=1, keepdims=True)

    b = dist * mask
    top_b, _ = _top5(b)
    sum_b = jnp.sum(b, axis=1, keepdims=True)
    klab_ref[...] = top_b / (sum_b + 1e-12)
    kalt_ref[...] = top_b / sum_b


def _pass2_kernel(x_ref, r_ref, o_ref):
    x = x_ref[0]
    e = jnp.exp(x)
    tot = jnp.sum(e, axis=0)
    r = r_ref[0]
    o_ref[0] = e * (1.0 / tot) + r


def kernel(logits, distribution):
    N, C, H, W = logits.shape
    BH = 16
    HB = H // BH

    sums = pl.pallas_call(
        _pass1_kernel,
        grid=(N, HB),
        in_specs=[pl.BlockSpec((1, C, BH, W), lambda n, hb: (n, 0, hb, 0))],
        out_specs=pl.BlockSpec((1, 1, C), lambda n, hb: (n, 0, 0)),
        out_shape=jax.ShapeDtypeStruct((N, 1, C), jnp.float32),
        compiler_params=pltpu.CompilerParams(
            dimension_semantics=("parallel", "arbitrary")),
    )(logits)

    dist2 = distribution.reshape(N, C)
    r, ksm, klab, kalt = pl.pallas_call(
        functools.partial(_mask_kernel, 1.0 / (H * W)),
        in_specs=[
            pl.BlockSpec((N, 1, C), lambda: (0, 0, 0)),
            pl.BlockSpec((N, C), lambda: (0, 0)),
        ],
        out_specs=[
            pl.BlockSpec((N, 1, C), lambda: (0, 0, 0)),
            pl.BlockSpec((N, _TOP_K), lambda: (0, 0)),
            pl.BlockSpec((N, _TOP_K), lambda: (0, 0)),
            pl.BlockSpec((N, _TOP_K), lambda: (0, 0)),
        ],
        out_shape=[
            jax.ShapeDtypeStruct((N, 1, C), jnp.float32),
            jax.ShapeDtypeStruct((N, _TOP_K), jnp.float32),
            jax.ShapeDtypeStruct((N, _TOP_K), jnp.float32),
            jax.ShapeDtypeStruct((N, _TOP_K), jnp.float32),
        ],
    )(sums, dist2)

    r4 = r.reshape(N, C, 1, 1)
    corrected = pl.pallas_call(
        _pass2_kernel,
        grid=(N, HB),
        in_specs=[
            pl.BlockSpec((1, C, BH, W), lambda n, hb: (n, 0, hb, 0)),
            pl.BlockSpec((1, C, 1, 1), lambda n, hb: (n, 0, 0, 0)),
        ],
        out_specs=pl.BlockSpec((1, C, BH, W), lambda n, hb: (n, 0, hb, 0)),
        out_shape=jax.ShapeDtypeStruct((N, C, H, W), jnp.float32),
        compiler_params=pltpu.CompilerParams(
            dimension_semantics=("parallel", "parallel")),
    )(logits, r4)

    k1 = klab.reshape(N, _TOP_K, 1, 1)
    k2 = ksm.reshape(N, _TOP_K, 1, 1)
    k3 = kalt.reshape(N, _TOP_K, 1, 1)
    return (corrected, k1, k2, k3)


# speedup vs baseline: 1.6209x; 1.6209x over previous
import functools

import jax
import jax.numpy as jnp
from jax.experimental import pallas as pl
from jax.experimental.pallas import tpu as pltpu

_TOP_K = 5


def _pass1_kernel(x_ref, s_ref):
    x = x_ref[0]
    e = jnp.exp(x)
    tot = jnp.sum(e, axis=0)
    p = e * (1.0 / tot)
    part = jnp.sum(p, axis=(1, 2))
    part = part.reshape(1, 1, -1)

    @pl.when(pl.program_id(1) == 0)
    def _init():
        s_ref[...] = part

    @pl.when(pl.program_id(1) != 0)
    def _acc():
        s_ref[...] += part


def _top5(x):
    vals = []
    iota = jax.lax.broadcasted_iota(jnp.int32, x.shape, 1)
    for _ in range(_TOP_K):
        m = jnp.max(x, axis=1, keepdims=True)
        vals.append(m)
        idx = jnp.min(jnp.where(x == m, iota, 2**30), axis=1, keepdims=True)
        x = jnp.where(iota == idx, -jnp.inf, x)
    return jnp.concatenate(vals, axis=1), vals[-1]


def _mask_kernel(inv_hw, s_ref, d_ref, r_ref, ksm_ref, klab_ref, kalt_ref):
    sd = s_ref[:, 0, :] * inv_hw
    dist = d_ref[...]
    _, thresh = _top5(sd)
    mask = (sd > thresh).astype(jnp.float32)
    r_ref[...] = ((dist - sd) * mask)[:, None, :]

    a = sd * mask
    top_a, _ = _top5(a)
    ksm_ref[...] = top_a / jnp.sum(a, axis=1, keepdims=True)

    b = dist * mask
    top_b, _ = _top5(b)
    sum_b = jnp.sum(b, axis=1, keepdims=True)
    klab_ref[...] = top_b / (sum_b + 1e-12)
    kalt_ref[...] = top_b / sum_b


def _pass2_kernel(x_ref, r_ref, o_ref):
    x = x_ref[0]
    e = jnp.exp(x)
    tot = jnp.sum(e, axis=0)
    r = r_ref[0]
    o_ref[0] = e * (1.0 / tot) + r


def kernel(logits, distribution):
    N, C, H, W = logits.shape
    BH = 64
    HB = H // BH

    sums = pl.pallas_call(
        _pass1_kernel,
        grid=(N, HB),
        in_specs=[pl.BlockSpec((1, C, BH, W), lambda n, hb: (n, 0, hb, 0))],
        out_specs=pl.BlockSpec((1, 1, C), lambda n, hb: (n, 0, 0)),
        out_shape=jax.ShapeDtypeStruct((N, 1, C), jnp.float32),
        compiler_params=pltpu.CompilerParams(
            dimension_semantics=("parallel", "arbitrary")),
    )(logits)

    dist2 = distribution.reshape(N, C)
    r, ksm, klab, kalt = pl.pallas_call(
        functools.partial(_mask_kernel, 1.0 / (H * W)),
        in_specs=[
            pl.BlockSpec((N, 1, C), lambda: (0, 0, 0)),
            pl.BlockSpec((N, C), lambda: (0, 0)),
        ],
        out_specs=[
            pl.BlockSpec((N, 1, C), lambda: (0, 0, 0)),
            pl.BlockSpec((N, _TOP_K), lambda: (0, 0)),
            pl.BlockSpec((N, _TOP_K), lambda: (0, 0)),
            pl.BlockSpec((N, _TOP_K), lambda: (0, 0)),
        ],
        out_shape=[
            jax.ShapeDtypeStruct((N, 1, C), jnp.float32),
            jax.ShapeDtypeStruct((N, _TOP_K), jnp.float32),
            jax.ShapeDtypeStruct((N, _TOP_K), jnp.float32),
            jax.ShapeDtypeStruct((N, _TOP_K), jnp.float32),
        ],
    )(sums, dist2)

    r4 = r.reshape(N, C, 1, 1)
    corrected = pl.pallas_call(
        _pass2_kernel,
        grid=(N, HB),
        in_specs=[
            pl.BlockSpec((1, C, BH, W), lambda n, hb: (n, 0, hb, 0)),
            pl.BlockSpec((1, C, 1, 1), lambda n, hb: (n, 0, 0, 0)),
        ],
        out_specs=pl.BlockSpec((1, C, BH, W), lambda n, hb: (n, 0, hb, 0)),
        out_shape=jax.ShapeDtypeStruct((N, C, H, W), jnp.float32),
        compiler_params=pltpu.CompilerParams(
            dimension_semantics=("parallel", "parallel")),
    )(logits, r4)

    k1 = klab.reshape(N, _TOP_K, 1, 1)
    k2 = ksm.reshape(N, _TOP_K, 1, 1)
    k3 = kalt.reshape(N, _TOP_K, 1, 1)
    return (corrected, k1, k2, k3)


# speedup vs baseline: 1.7298x; 1.0672x over previous
import functools

import jax
import jax.numpy as jnp
from jax.experimental import pallas as pl
from jax.experimental.pallas import tpu as pltpu

_TOP_K = 5


def _pass1_kernel(x_ref, s_ref):
    x = x_ref[0]
    e = jnp.exp(x)
    tot = jnp.sum(e, axis=0)
    p = e * (1.0 / tot)
    part = jnp.sum(p, axis=(1, 2))
    part = part.reshape(1, 1, -1)

    @pl.when(pl.program_id(1) == 0)
    def _init():
        s_ref[...] = part

    @pl.when(pl.program_id(1) != 0)
    def _acc():
        s_ref[...] += part


def _top5(x):
    vals = []
    iota = jax.lax.broadcasted_iota(jnp.int32, x.shape, 1)
    for _ in range(_TOP_K):
        m = jnp.max(x, axis=1, keepdims=True)
        vals.append(m)
        idx = jnp.min(jnp.where(x == m, iota, 2**30), axis=1, keepdims=True)
        x = jnp.where(iota == idx, -jnp.inf, x)
    return jnp.concatenate(vals, axis=1), vals[-1]


def _mask_kernel(inv_hw, s_ref, d_ref, r_ref, ksm_ref, klab_ref, kalt_ref):
    sd = s_ref[:, 0, :] * inv_hw
    dist = d_ref[...]
    _, thresh = _top5(sd)
    mask = (sd > thresh).astype(jnp.float32)
    r_ref[...] = ((dist - sd) * mask)[:, None, :]

    a = sd * mask
    top_a, _ = _top5(a)
    ksm_ref[...] = top_a / jnp.sum(a, axis=1, keepdims=True)

    b = dist * mask
    top_b, _ = _top5(b)
    sum_b = jnp.sum(b, axis=1, keepdims=True)
    klab_ref[...] = top_b / (sum_b + 1e-12)
    kalt_ref[...] = top_b / sum_b


def _pass2_kernel(x_ref, r_ref, o_ref):
    x = x_ref[0]
    e = jnp.exp(x)
    tot = jnp.sum(e, axis=0)
    r = r_ref[0]
    o_ref[0] = e * (1.0 / tot) + r


def kernel(logits, distribution):
    N, C, H, W = logits.shape
    BH = 128
    HB = H // BH

    sums = pl.pallas_call(
        _pass1_kernel,
        grid=(N, HB),
        in_specs=[pl.BlockSpec((1, C, BH, W), lambda n, hb: (n, 0, hb, 0))],
        out_specs=pl.BlockSpec((1, 1, C), lambda n, hb: (n, 0, 0)),
        out_shape=jax.ShapeDtypeStruct((N, 1, C), jnp.float32),
        compiler_params=pltpu.CompilerParams(
            dimension_semantics=("parallel", "arbitrary")),
    )(logits)

    dist2 = distribution.reshape(N, C)
    r, ksm, klab, kalt = pl.pallas_call(
        functools.partial(_mask_kernel, 1.0 / (H * W)),
        in_specs=[
            pl.BlockSpec((N, 1, C), lambda: (0, 0, 0)),
            pl.BlockSpec((N, C), lambda: (0, 0)),
        ],
        out_specs=[
            pl.BlockSpec((N, 1, C), lambda: (0, 0, 0)),
            pl.BlockSpec((N, _TOP_K), lambda: (0, 0)),
            pl.BlockSpec((N, _TOP_K), lambda: (0, 0)),
            pl.BlockSpec((N, _TOP_K), lambda: (0, 0)),
        ],
        out_shape=[
            jax.ShapeDtypeStruct((N, 1, C), jnp.float32),
            jax.ShapeDtypeStruct((N, _TOP_K), jnp.float32),
            jax.ShapeDtypeStruct((N, _TOP_K), jnp.float32),
            jax.ShapeDtypeStruct((N, _TOP_K), jnp.float32),
        ],
    )(sums, dist2)

    r4 = r.reshape(N, C, 1, 1)
    corrected = pl.pallas_call(
        _pass2_kernel,
        grid=(N, HB),
        in_specs=[
            pl.BlockSpec((1, C, BH, W), lambda n, hb: (n, 0, hb, 0)),
            pl.BlockSpec((1, C, 1, 1), lambda n, hb: (n, 0, 0, 0)),
        ],
        out_specs=pl.BlockSpec((1, C, BH, W), lambda n, hb: (n, 0, hb, 0)),
        out_shape=jax.ShapeDtypeStruct((N, C, H, W), jnp.float32),
        compiler_params=pltpu.CompilerParams(
            dimension_semantics=("parallel", "parallel")),
    )(logits, r4)

    k1 = klab.reshape(N, _TOP_K, 1, 1)
    k2 = ksm.reshape(N, _TOP_K, 1, 1)
    k3 = kalt.reshape(N, _TOP_K, 1, 1)
    return (corrected, k1, k2, k3)


# speedup vs baseline: 2.4285x; 1.4040x over previous
import functools

import jax
import jax.numpy as jnp
from jax.experimental import pallas as pl
from jax.experimental.pallas import tpu as pltpu

_TOP_K = 5


def _top5_cmajor(v):
    iota = jax.lax.broadcasted_iota(jnp.int32, v.shape, 0)
    x = v
    ms = []
    for _ in range(_TOP_K):
        m = jnp.max(x, axis=0, keepdims=True)
        ms.append(m)
        idx = jnp.min(jnp.where(x == m, iota, 2**30), axis=0, keepdims=True)
        x = jnp.where(iota == idx, -jnp.inf, x)
    return ms


def _lanes5(ms):
    li = jax.lax.broadcasted_iota(jnp.int32, (1, 1, _TOP_K), 2)
    out = jnp.zeros((1, 1, _TOP_K), jnp.float32)
    for i in range(_TOP_K):
        out = jnp.where(li == i, ms[i], out)
    return out


def _fused_kernel(inv_hw, x_ref, d_ref, o_ref, ksm_ref, klab_ref, kalt_ref):
    x = x_ref[0]
    e = jnp.exp(x)
    tot = jnp.sum(e, axis=0)
    o_ref[0] = e * (1.0 / tot)
    sd = jnp.sum(o_ref[0], axis=(1, 2), keepdims=True) * inv_hw
    dist = d_ref[0]

    thresh = _top5_cmajor(sd)[-1]
    mask = (sd > thresh).astype(jnp.float32)
    r = (dist - sd) * mask
    o_ref[0] = o_ref[0] + r

    a = sd * mask
    ksm_ref[...] = (_lanes5(_top5_cmajor(a))
                    / jnp.sum(a, axis=0, keepdims=True))
    b = dist * mask
    sum_b = jnp.sum(b, axis=0, keepdims=True)
    top_b = _lanes5(_top5_cmajor(b))
    klab_ref[...] = top_b / (sum_b + 1e-12)
    kalt_ref[...] = top_b / sum_b


def kernel(logits, distribution):
    N, C, H, W = logits.shape

    corrected, ksm, klab, kalt = pl.pallas_call(
        functools.partial(_fused_kernel, 1.0 / (H * W)),
        grid=(N,),
        in_specs=[
            pl.BlockSpec((1, C, H, W), lambda n: (n, 0, 0, 0)),
            pl.BlockSpec((1, C, 1, 1), lambda n: (n, 0, 0, 0)),
        ],
        out_specs=[
            pl.BlockSpec((1, C, H, W), lambda n: (n, 0, 0, 0)),
            pl.BlockSpec((1, 1, _TOP_K), lambda n: (n, 0, 0)),
            pl.BlockSpec((1, 1, _TOP_K), lambda n: (n, 0, 0)),
            pl.BlockSpec((1, 1, _TOP_K), lambda n: (n, 0, 0)),
        ],
        out_shape=[
            jax.ShapeDtypeStruct((N, C, H, W), jnp.float32),
            jax.ShapeDtypeStruct((N, 1, _TOP_K), jnp.float32),
            jax.ShapeDtypeStruct((N, 1, _TOP_K), jnp.float32),
            jax.ShapeDtypeStruct((N, 1, _TOP_K), jnp.float32),
        ],
        compiler_params=pltpu.CompilerParams(
            dimension_semantics=("parallel",)),
    )(logits, distribution)

    k1 = klab.reshape(N, _TOP_K, 1, 1)
    k2 = ksm.reshape(N, _TOP_K, 1, 1)
    k3 = kalt.reshape(N, _TOP_K, 1, 1)
    return (corrected, k1, k2, k3)


# speedup vs baseline: 2.5720x; 1.0591x over previous
import functools

import jax
import jax.numpy as jnp
from jax.experimental import pallas as pl
from jax.experimental.pallas import tpu as pltpu

_TOP_K = 5


def _top5_lanes(v):
    iota = jax.lax.broadcasted_iota(jnp.int32, v.shape, 2)
    x = v
    ms = []
    for _ in range(_TOP_K):
        m = jnp.max(x, axis=2, keepdims=True)
        ms.append(m)
        idx = jnp.min(jnp.where(x == m, iota, 2**30), axis=2, keepdims=True)
        x = jnp.where(iota == idx, -jnp.inf, x)
    return ms


def _lanes5(ms):
    li = jax.lax.broadcasted_iota(jnp.int32, (1, 1, _TOP_K), 2)
    out = jnp.zeros((1, 1, _TOP_K), jnp.float32)
    for i in range(_TOP_K):
        out = jnp.where(li == i, ms[i], out)
    return out


def _fused_kernel(inv_hw, x_ref, d_ref, o_ref, ksm_ref, klab_ref, kalt_ref):
    C = x_ref.shape[1]
    x = x_ref[0]
    e = jnp.exp(x)
    tot = jnp.sum(e, axis=0)
    recip = 1.0 / tot
    sd = jnp.sum(e * recip, axis=(1, 2), keepdims=True) * inv_hw

    sdl = sd.reshape(1, 1, C)
    dist = d_ref[...]
    ms = _top5_lanes(sdl)
    thresh = ms[-1]
    mask = (sdl > thresh).astype(jnp.float32)
    rl = (dist - sdl) * mask
    r = rl.reshape(C, 1, 1)

    o_ref[0] = e * recip + r

    sum_a = jnp.sum(sdl * mask, axis=2, keepdims=True)
    top_a = _lanes5([jnp.where(m > thresh, m, 0.0) for m in ms])
    ksm_ref[...] = top_a / sum_a

    b = dist * mask
    sum_b = jnp.sum(b, axis=2, keepdims=True)
    top_b = _lanes5(_top5_lanes(b))
    klab_ref[...] = top_b / (sum_b + 1e-12)
    kalt_ref[...] = top_b / sum_b


def kernel(logits, distribution):
    N, C, H, W = logits.shape

    dist_l = distribution.reshape(N, 1, C)
    corrected, ksm, klab, kalt = pl.pallas_call(
        functools.partial(_fused_kernel, 1.0 / (H * W)),
        grid=(N,),
        in_specs=[
            pl.BlockSpec((1, C, H, W), lambda n: (n, 0, 0, 0)),
            pl.BlockSpec((1, 1, C), lambda n: (n, 0, 0)),
        ],
        out_specs=[
            pl.BlockSpec((1, C, H, W), lambda n: (n, 0, 0, 0)),
            pl.BlockSpec((1, 1, _TOP_K), lambda n: (n, 0, 0)),
            pl.BlockSpec((1, 1, _TOP_K), lambda n: (n, 0, 0)),
            pl.BlockSpec((1, 1, _TOP_K), lambda n: (n, 0, 0)),
        ],
        out_shape=[
            jax.ShapeDtypeStruct((N, C, H, W), jnp.float32),
            jax.ShapeDtypeStruct((N, 1, _TOP_K), jnp.float32),
            jax.ShapeDtypeStruct((N, 1, _TOP_K), jnp.float32),
            jax.ShapeDtypeStruct((N, 1, _TOP_K), jnp.float32),
        ],
        compiler_params=pltpu.CompilerParams(
            dimension_semantics=("parallel",)),
    )(logits, dist_l)

    k1 = klab.reshape(N, _TOP_K, 1, 1)
    k2 = ksm.reshape(N, _TOP_K, 1, 1)
    k3 = kalt.reshape(N, _TOP_K, 1, 1)
    return (corrected, k1, k2, k3)
